# baseline (speedup 1.0000x reference)
import functools

import jax
import jax.numpy as jnp
from jax import lax
from jax.experimental import pallas as pl
from jax.experimental.pallas import tpu as pltpu
from jax.experimental.pallas import tpu_sc as plsc

N = 4096
C_IN = 4
C_OUT = 2
E = 131072

NC = 2
NS = 16
L = 16

R = 128
SLAB = R * N
N_SLABS = N // R
ET = E // NS
ROWS128 = ET // 128
ZB = 4096
WB = SLAB // NS


def _coalesce_body(ei_hbm, ev_hbm, d_hbm,
                   row_b, col_b, g_b, v_b, idx_st, val_st, zero_b, dummy_b,
                   slab0, slab1, wb_sem0, wb_sem1, sc_sem, clr_sem):
    cid = lax.axis_index("c")
    sid = lax.axis_index("s")
    slabs = (slab0, slab1)
    wb_sems = (wb_sem0, wb_sem1)

    def _zero(i, _):
        zero_b[pl.ds(i * L, L)] = jnp.zeros((L,), jnp.float32)
        return 0
    lax.fori_loop(0, ZB // L, _zero, 0)

    def _wb_wait(half):
        pltpu.make_async_copy(slabs[half].at[pl.ds(sid * WB, WB)],
                              d_hbm.at[pl.ds(sid * WB, WB)],
                              wb_sems[half]).wait()

    for ch_local in range(C_IN // NC):
        ch = cid * (C_IN // NC) + ch_local
        ebase = ch * 2 * E + sid * ET
        pltpu.sync_copy(ei_hbm.at[pl.ds(ebase, ET)], row_b)
        pltpu.sync_copy(ei_hbm.at[pl.ds(ebase + E, ET)], col_b)
        pltpu.sync_copy(ev_hbm.at[pl.ds(ch * E + sid * ET, ET)], v_b)

        def _flat(i, _):
            r = jnp.minimum(row_b[pl.ds(i * L, L)], N - 1)
            c = jnp.minimum(col_b[pl.ds(i * L, L)], N - 1)
            g_b[pl.ds(i * L, L)] = r * N + c
            return 0
        lax.fori_loop(0, ET // L, _flat, 0)

        def _slab_pair(s2, _):
            for half in range(2):
                s = s2 * 2 + half
                slab = slabs[half]

                if ch_local == 0:
                    @pl.when(s2 > 0)
                    def _():
                        _wb_wait(half)
                else:
                    _wb_wait(half)

                clears = [pltpu.async_copy(
                    zero_b, slab.at[pl.ds(sid * WB + k * ZB, ZB)], clr_sem)
                    for k in range(WB // ZB)]

                lo = s * SLAB

                def _stage(r128, _):
                    for k in range(128 // L):
                        i = r128 * (128 // L) + k
                        g = g_b[pl.ds(i * L, L)]
                        local = g - lo
                        inb = (local >= 0) & (local < SLAB)
                        idx_st[r128, pl.ds(k * L, L)] = lax.bitwise_and(
                            local, SLAB - 1)
                        v = v_b[pl.ds(i * L, L)]
                        val_st[r128, pl.ds(k * L, L)] = jnp.where(inb, v, 0.0)
                    return 0
                lax.fori_loop(0, ROWS128, _stage, 0)
                for cp in clears:
                    cp.wait()
                plsc.subcore_barrier()

                def _scat(r128, _):
                    pltpu.async_copy(val_st.at[r128],
                                     slab.at[idx_st.at[r128]], sc_sem,
                                     add=True)
                    return 0
                lax.fori_loop(0, ROWS128, _scat, 0)
                pltpu.make_async_copy(ei_hbm.at[pl.ds(0, ET)], dummy_b,
                                      sc_sem).wait()
                plsc.subcore_barrier()

                dst = ch * N * N + s * SLAB + sid * WB
                pltpu.async_copy(slab.at[pl.ds(sid * WB, WB)],
                                 d_hbm.at[pl.ds(dst, WB)], wb_sems[half])
            return 0
        lax.fori_loop(0, N_SLABS // 2, _slab_pair, 0)
    for half in range(2):
        _wb_wait(half)


def _coalesce(edge_index, edge_value):
    mesh = plsc.VectorSubcoreMesh(core_axis_name="c", subcore_axis_name="s")
    kern = pl.kernel(
        _coalesce_body,
        out_type=jax.ShapeDtypeStruct((C_IN * N * N,), jnp.float32),
        mesh=mesh,
        scratch_types=[
            pltpu.VMEM((ET,), jnp.int32),
            pltpu.VMEM((ET,), jnp.int32),
            pltpu.VMEM((ET,), jnp.int32),
            pltpu.VMEM((ET,), jnp.float32),
            pltpu.VMEM((ROWS128, 128), jnp.int32),
            pltpu.VMEM((ROWS128, 128), jnp.float32),
            pltpu.VMEM((ZB,), jnp.float32),
            pltpu.VMEM((ET,), jnp.int32),
            pltpu.VMEM_SHARED((SLAB,), jnp.float32),
            pltpu.VMEM_SHARED((SLAB,), jnp.float32),
            pltpu.SemaphoreType.DMA,
            pltpu.SemaphoreType.DMA,
            pltpu.SemaphoreType.DMA,
            pltpu.SemaphoreType.DMA,
        ],
    )
    return kern(edge_index.reshape(-1), edge_value.reshape(-1))


MIX_BM = 256


def _mix_body(f1_ref, f2_ref, d_ref, a_ref, b_ref):
    d = d_ref[...]
    for c in range(C_OUT):
        acc_a = f1_ref[c, 0] * d[0]
        acc_b = f2_ref[c, 0] * d[0]
        for j in range(1, C_IN):
            acc_a += f1_ref[c, j] * d[j]
            acc_b += f2_ref[c, j] * d[j]
        a_ref[c] = acc_a.astype(jnp.bfloat16)
        b_ref[c] = acc_b.astype(jnp.bfloat16)


def _mix(filt1, filt2, d):
    grid = (N // MIX_BM,)
    return pl.pallas_call(
        _mix_body,
        grid=grid,
        in_specs=[
            pl.BlockSpec(memory_space=pltpu.SMEM),
            pl.BlockSpec(memory_space=pltpu.SMEM),
            pl.BlockSpec((C_IN, MIX_BM, N), lambda i: (0, i, 0)),
        ],
        out_specs=[
            pl.BlockSpec((C_OUT, MIX_BM, N), lambda i: (0, i, 0)),
            pl.BlockSpec((C_OUT, MIX_BM, N), lambda i: (0, i, 0)),
        ],
        out_shape=[
            jax.ShapeDtypeStruct((C_OUT, N, N), jnp.bfloat16),
            jax.ShapeDtypeStruct((C_OUT, N, N), jnp.bfloat16),
        ],
        compiler_params=pltpu.CompilerParams(
            dimension_semantics=("arbitrary",),
        ),
    )(filt1, filt2, d)


BM = 2048
BN = 2048
BK = 1024


def _mm_body(a_ref, b_ref, h_ref):
    k = pl.program_id(3)

    @pl.when(k == 0)
    def _():
        h_ref[...] = jnp.zeros_like(h_ref)

    h_ref[0] += jnp.dot(a_ref[0], b_ref[0], preferred_element_type=jnp.float32)


def _matmul(a, b):
    grid = (C_OUT, N // BM, N // BN, N // BK)
    return pl.pallas_call(
        _mm_body,
        grid=grid,
        in_specs=[
            pl.BlockSpec((1, BM, BK), lambda c, i, j, k: (c, i, k)),
            pl.BlockSpec((1, BK, BN), lambda c, i, j, k: (c, k, j)),
        ],
        out_specs=pl.BlockSpec((1, BM, BN), lambda c, i, j, k: (c, i, j)),
        out_shape=jax.ShapeDtypeStruct((C_OUT, N, N), jnp.float32),
        compiler_params=pltpu.CompilerParams(
            dimension_semantics=("parallel", "parallel", "parallel", "arbitrary"),
        ),
    )(a, b)


def kernel(edge_index, edge_value, num_nodes, W1, W2):
    filt1 = jax.nn.softmax(W1, axis=1)
    filt2 = jax.nn.softmax(W2, axis=1)
    d = _coalesce(edge_index, edge_value).reshape(C_IN, N, N)
    a, b = _mix(filt1, filt2, d)
    h = _matmul(a, b)
    return (h, filt1, filt2)


# speedup vs baseline: 1.0134x; 1.0134x over previous
import functools

import jax
import jax.numpy as jnp
from jax import lax
from jax.experimental import pallas as pl
from jax.experimental.pallas import tpu as pltpu
from jax.experimental.pallas import tpu_sc as plsc

N = 4096
C_IN = 4
C_OUT = 2
E = 131072

NC = 2
NS = 16
L = 16

R = 128
SLAB = R * N
N_SLABS = N // R
ET = E // NS
ROWS128 = ET // 128
ZB = 4096
WB = SLAB // NS


def _coalesce_body(ei_hbm, ev_hbm, d_hbm,
                   row_b, col_b, g_b, v_b, idx_st, val_st, zero_b, dummy_b,
                   slab0, slab1, wb_sem0, wb_sem1, sc_sem, clr_sem):
    cid = lax.axis_index("c")
    sid = lax.axis_index("s")
    slabs = (slab0, slab1)
    wb_sems = (wb_sem0, wb_sem1)

    def _zero(i, _):
        zero_b[pl.ds(i * L, L)] = jnp.zeros((L,), jnp.float32)
        return 0
    lax.fori_loop(0, ZB // L, _zero, 0)

    def _wb_wait(half):
        pltpu.make_async_copy(slabs[half].at[pl.ds(sid * WB, WB)],
                              d_hbm.at[pl.ds(sid * WB, WB)],
                              wb_sems[half]).wait()

    def _fire_clears(half):
        for k in range(WB // ZB):
            pltpu.async_copy(zero_b, slabs[half].at[pl.ds(sid * WB + k * ZB,
                                                          ZB)], clr_sem)

    def _wait_clears(half):
        for k in range(WB // ZB):
            pltpu.make_async_copy(
                zero_b, slabs[half].at[pl.ds(sid * WB + k * ZB, ZB)],
                clr_sem).wait()

    _fire_clears(0)

    for ch_local in range(C_IN // NC):
        ch = cid * (C_IN // NC) + ch_local
        ebase = ch * 2 * E + sid * ET
        pltpu.sync_copy(ei_hbm.at[pl.ds(ebase, ET)], row_b)
        pltpu.sync_copy(ei_hbm.at[pl.ds(ebase + E, ET)], col_b)
        pltpu.sync_copy(ev_hbm.at[pl.ds(ch * E + sid * ET, ET)], v_b)

        def _flat(i, _):
            r = jnp.minimum(row_b[pl.ds(i * L, L)], N - 1)
            c = jnp.minimum(col_b[pl.ds(i * L, L)], N - 1)
            g_b[pl.ds(i * L, L)] = r * N + c
            return 0
        lax.fori_loop(0, ET // L, _flat, 0)

        def _slab_pair(s2, _):
            for half in range(2):
                s = s2 * 2 + half
                slab = slabs[half]

                _wait_clears(half)
                plsc.subcore_barrier()

                lo = s * SLAB

                def _stage(r128, _):
                    for k in range(128 // L):
                        i = r128 * (128 // L) + k
                        g = g_b[pl.ds(i * L, L)]
                        local = g - lo
                        inb = (local >= 0) & (local < SLAB)
                        idx_st[r128, pl.ds(k * L, L)] = lax.bitwise_and(
                            local, SLAB - 1)
                        v = v_b[pl.ds(i * L, L)]
                        val_st[r128, pl.ds(k * L, L)] = jnp.where(inb, v, 0.0)
                    pltpu.async_copy(val_st.at[r128],
                                     slab.at[idx_st.at[r128]], sc_sem,
                                     add=True)
                    return 0
                lax.fori_loop(0, ROWS128, _stage, 0)
                pltpu.make_async_copy(ei_hbm.at[pl.ds(0, ET)], dummy_b,
                                      sc_sem).wait()
                plsc.subcore_barrier()

                dst = ch * N * N + s * SLAB + sid * WB
                pltpu.async_copy(slab.at[pl.ds(sid * WB, WB)],
                                 d_hbm.at[pl.ds(dst, WB)], wb_sems[half])

                other = 1 - half
                if ch_local == 0 and half == 0:
                    @pl.when(s2 > 0)
                    def _():
                        _wb_wait(other)
                else:
                    _wb_wait(other)
                _fire_clears(other)
            return 0
        lax.fori_loop(0, N_SLABS // 2, _slab_pair, 0)
    _wait_clears(0)
    _wb_wait(1)


def _coalesce(edge_index, edge_value):
    mesh = plsc.VectorSubcoreMesh(core_axis_name="c", subcore_axis_name="s")
    kern = pl.kernel(
        _coalesce_body,
        out_type=jax.ShapeDtypeStruct((C_IN * N * N,), jnp.float32),
        mesh=mesh,
        scratch_types=[
            pltpu.VMEM((ET,), jnp.int32),
            pltpu.VMEM((ET,), jnp.int32),
            pltpu.VMEM((ET,), jnp.int32),
            pltpu.VMEM((ET,), jnp.float32),
            pltpu.VMEM((ROWS128, 128), jnp.int32),
            pltpu.VMEM((ROWS128, 128), jnp.float32),
            pltpu.VMEM((ZB,), jnp.float32),
            pltpu.VMEM((ET,), jnp.int32),
            pltpu.VMEM_SHARED((SLAB,), jnp.float32),
            pltpu.VMEM_SHARED((SLAB,), jnp.float32),
            pltpu.SemaphoreType.DMA,
            pltpu.SemaphoreType.DMA,
            pltpu.SemaphoreType.DMA,
            pltpu.SemaphoreType.DMA,
        ],
    )
    return kern(edge_index.reshape(-1), edge_value.reshape(-1))


MIX_BM = 256


def _mix_body(f1_ref, f2_ref, d_ref, a_ref, b_ref):
    d = d_ref[...]
    for c in range(C_OUT):
        acc_a = f1_ref[c, 0] * d[0]
        acc_b = f2_ref[c, 0] * d[0]
        for j in range(1, C_IN):
            acc_a += f1_ref[c, j] * d[j]
            acc_b += f2_ref[c, j] * d[j]
        a_ref[c] = acc_a.astype(jnp.bfloat16)
        b_ref[c] = acc_b.astype(jnp.bfloat16)


def _mix(filt1, filt2, d):
    grid = (N // MIX_BM,)
    return pl.pallas_call(
        _mix_body,
        grid=grid,
        in_specs=[
            pl.BlockSpec(memory_space=pltpu.SMEM),
            pl.BlockSpec(memory_space=pltpu.SMEM),
            pl.BlockSpec((C_IN, MIX_BM, N), lambda i: (0, i, 0)),
        ],
        out_specs=[
            pl.BlockSpec((C_OUT, MIX_BM, N), lambda i: (0, i, 0)),
            pl.BlockSpec((C_OUT, MIX_BM, N), lambda i: (0, i, 0)),
        ],
        out_shape=[
            jax.ShapeDtypeStruct((C_OUT, N, N), jnp.bfloat16),
            jax.ShapeDtypeStruct((C_OUT, N, N), jnp.bfloat16),
        ],
        compiler_params=pltpu.CompilerParams(
            dimension_semantics=("arbitrary",),
        ),
    )(filt1, filt2, d)


BM = 2048
BN = 2048
BK = 1024


def _mm_body(a_ref, b_ref, h_ref):
    k = pl.program_id(3)

    @pl.when(k == 0)
    def _():
        h_ref[...] = jnp.zeros_like(h_ref)

    h_ref[0] += jnp.dot(a_ref[0], b_ref[0], preferred_element_type=jnp.float32)


def _matmul(a, b):
    grid = (C_OUT, N // BM, N // BN, N // BK)
    return pl.pallas_call(
        _mm_body,
        grid=grid,
        in_specs=[
            pl.BlockSpec((1, BM, BK), lambda c, i, j, k: (c, i, k)),
            pl.BlockSpec((1, BK, BN), lambda c, i, j, k: (c, k, j)),
        ],
        out_specs=pl.BlockSpec((1, BM, BN), lambda c, i, j, k: (c, i, j)),
        out_shape=jax.ShapeDtypeStruct((C_OUT, N, N), jnp.float32),
        compiler_params=pltpu.CompilerParams(
            dimension_semantics=("parallel", "parallel", "parallel", "arbitrary"),
        ),
    )(a, b)


def kernel(edge_index, edge_value, num_nodes, W1, W2):
    filt1 = jax.nn.softmax(W1, axis=1)
    filt2 = jax.nn.softmax(W2, axis=1)
    d = _coalesce(edge_index, edge_value).reshape(C_IN, N, N)
    a, b = _mix(filt1, filt2, d)
    h = _matmul(a, b)
    return (h, filt1, filt2)


# speedup vs baseline: 1.0144x; 1.0010x over previous
import functools

import jax
import jax.numpy as jnp
from jax import lax
from jax.experimental import pallas as pl
from jax.experimental.pallas import tpu as pltpu
from jax.experimental.pallas import tpu_sc as plsc

N = 4096
C_IN = 4
C_OUT = 2
E = 131072

NC = 2
NS = 16
L = 16

R = 128
SLAB = R * N
N_SLABS = N // R
ET = E // NS
ROWS128 = ET // 128
ZB = 4096
WB = SLAB // NS


def _coalesce_body(ei_hbm, ev_hbm, d_hbm,
                   row_b, col_b, g_b, v_b, idx_st, val_st, zero_b, dummy_b,
                   slab0, slab1, wb_sem0, wb_sem1, sc_sem, clr_sem):
    cid = lax.axis_index("c")
    sid = lax.axis_index("s")
    slabs = (slab0, slab1)
    wb_sems = (wb_sem0, wb_sem1)

    def _zero(i, _):
        zero_b[pl.ds(i * L, L)] = jnp.zeros((L,), jnp.float32)
        return 0
    lax.fori_loop(0, ZB // L, _zero, 0)

    def _wb_wait(half):
        pltpu.make_async_copy(slabs[half].at[pl.ds(sid * WB, WB)],
                              d_hbm.at[pl.ds(sid * WB, WB)],
                              wb_sems[half]).wait()

    def _fire_clears(half):
        for k in range(WB // ZB):
            pltpu.async_copy(zero_b, slabs[half].at[pl.ds(sid * WB + k * ZB,
                                                          ZB)], clr_sem)

    def _wait_clears(half):
        for k in range(WB // ZB):
            pltpu.make_async_copy(
                zero_b, slabs[half].at[pl.ds(sid * WB + k * ZB, ZB)],
                clr_sem).wait()

    _fire_clears(0)

    for ch_local in range(C_IN // NC):
        ch = cid * (C_IN // NC) + ch_local
        ebase = ch * 2 * E + sid * ET
        pltpu.sync_copy(ei_hbm.at[pl.ds(ebase, ET)], row_b)
        pltpu.sync_copy(ei_hbm.at[pl.ds(ebase + E, ET)], col_b)
        pltpu.sync_copy(ev_hbm.at[pl.ds(ch * E + sid * ET, ET)], v_b)

        def _flat(i, _):
            r = jnp.minimum(row_b[pl.ds(i * L, L)], N - 1)
            c = jnp.minimum(col_b[pl.ds(i * L, L)], N - 1)
            g_b[pl.ds(i * L, L)] = r * N + c
            return 0
        lax.fori_loop(0, ET // L, _flat, 0)

        def _slab_pair(s2, _):
            for half in range(2):
                s = s2 * 2 + half
                slab = slabs[half]

                _wait_clears(half)
                plsc.subcore_barrier()

                lo = s * SLAB

                def _stage(r128, _):
                    for k in range(128 // L):
                        i = r128 * (128 // L) + k
                        g = g_b[pl.ds(i * L, L)]
                        local = g - lo
                        inb = (local >= 0) & (local < SLAB)
                        idx_st[r128, pl.ds(k * L, L)] = lax.bitwise_and(
                            local, SLAB - 1)
                        v = v_b[pl.ds(i * L, L)]
                        val_st[r128, pl.ds(k * L, L)] = jnp.where(inb, v, 0.0)
                    pltpu.async_copy(val_st.at[r128],
                                     slab.at[idx_st.at[r128]], sc_sem,
                                     add=True)
                    return 0
                lax.fori_loop(0, ROWS128, _stage, 0)
                pltpu.make_async_copy(ei_hbm.at[pl.ds(0, ET)], dummy_b,
                                      sc_sem).wait()
                plsc.subcore_barrier()

                dst = ch * N * N + s * SLAB + sid * WB
                pltpu.async_copy(slab.at[pl.ds(sid * WB, WB)],
                                 d_hbm.at[pl.ds(dst, WB)], wb_sems[half])

                other = 1 - half
                if ch_local == 0 and half == 0:
                    @pl.when(s2 > 0)
                    def _():
                        _wb_wait(other)
                else:
                    _wb_wait(other)
                _fire_clears(other)
            return 0
        lax.fori_loop(0, N_SLABS // 2, _slab_pair, 0)
    _wait_clears(0)
    _wb_wait(1)


def _coalesce(edge_index, edge_value):
    mesh = plsc.VectorSubcoreMesh(core_axis_name="c", subcore_axis_name="s")
    kern = pl.kernel(
        _coalesce_body,
        out_type=jax.ShapeDtypeStruct((C_IN * N * N,), jnp.float32),
        mesh=mesh,
        scratch_types=[
            pltpu.VMEM((ET,), jnp.int32),
            pltpu.VMEM((ET,), jnp.int32),
            pltpu.VMEM((ET,), jnp.int32),
            pltpu.VMEM((ET,), jnp.float32),
            pltpu.VMEM((ROWS128, 128), jnp.int32),
            pltpu.VMEM((ROWS128, 128), jnp.float32),
            pltpu.VMEM((ZB,), jnp.float32),
            pltpu.VMEM((ET,), jnp.int32),
            pltpu.VMEM_SHARED((SLAB,), jnp.float32),
            pltpu.VMEM_SHARED((SLAB,), jnp.float32),
            pltpu.SemaphoreType.DMA,
            pltpu.SemaphoreType.DMA,
            pltpu.SemaphoreType.DMA,
            pltpu.SemaphoreType.DMA,
        ],
    )
    return kern(edge_index.reshape(-1), edge_value.reshape(-1))


MIX_BM = 256


def _mix_body(f1_ref, f2_ref, d_ref, a_ref, b_ref):
    d = d_ref[...]
    for c in range(C_OUT):
        acc_a = f1_ref[c, 0] * d[0]
        acc_b = f2_ref[c, 0] * d[0]
        for j in range(1, C_IN):
            acc_a += f1_ref[c, j] * d[j]
            acc_b += f2_ref[c, j] * d[j]
        a_ref[c] = acc_a.astype(jnp.bfloat16)
        b_ref[c] = acc_b.astype(jnp.bfloat16)


def _mix(filt1, filt2, d):
    grid = (N // MIX_BM,)
    return pl.pallas_call(
        _mix_body,
        grid=grid,
        in_specs=[
            pl.BlockSpec(memory_space=pltpu.SMEM),
            pl.BlockSpec(memory_space=pltpu.SMEM),
            pl.BlockSpec((C_IN, MIX_BM, N), lambda i: (0, i, 0)),
        ],
        out_specs=[
            pl.BlockSpec((C_OUT, MIX_BM, N), lambda i: (0, i, 0)),
            pl.BlockSpec((C_OUT, MIX_BM, N), lambda i: (0, i, 0)),
        ],
        out_shape=[
            jax.ShapeDtypeStruct((C_OUT, N, N), jnp.bfloat16),
            jax.ShapeDtypeStruct((C_OUT, N, N), jnp.bfloat16),
        ],
        compiler_params=pltpu.CompilerParams(
            dimension_semantics=("parallel",),
        ),
    )(filt1, filt2, d)


BM = 2048
BN = 2048
BK = 1024


def _mm_body(a_ref, b_ref, h_ref):
    k = pl.program_id(3)

    @pl.when(k == 0)
    def _():
        h_ref[...] = jnp.zeros_like(h_ref)

    h_ref[0] += jnp.dot(a_ref[0], b_ref[0], preferred_element_type=jnp.float32)


def _matmul(a, b):
    grid = (C_OUT, N // BM, N // BN, N // BK)
    return pl.pallas_call(
        _mm_body,
        grid=grid,
        in_specs=[
            pl.BlockSpec((1, BM, BK), lambda c, i, j, k: (c, i, k)),
            pl.BlockSpec((1, BK, BN), lambda c, i, j, k: (c, k, j)),
        ],
        out_specs=pl.BlockSpec((1, BM, BN), lambda c, i, j, k: (c, i, j)),
        out_shape=jax.ShapeDtypeStruct((C_OUT, N, N), jnp.float32),
        compiler_params=pltpu.CompilerParams(
            dimension_semantics=("parallel", "parallel", "parallel", "arbitrary"),
        ),
    )(a, b)


def kernel(edge_index, edge_value, num_nodes, W1, W2):
    filt1 = jax.nn.softmax(W1, axis=1)
    filt2 = jax.nn.softmax(W2, axis=1)
    d = _coalesce(edge_index, edge_value).reshape(C_IN, N, N)
    a, b = _mix(filt1, filt2, d)
    h = _matmul(a, b)
    return (h, filt1, filt2)


# speedup vs baseline: 1.0145x; 1.0001x over previous
import jax
import jax.numpy as jnp
from jax import lax
from jax.experimental import pallas as pl
from jax.experimental.pallas import tpu as pltpu
from jax.experimental.pallas import tpu_sc as plsc

N = 4096
C_IN = 4
C_OUT = 2
E = 131072

NC = 2
NS = 16
L = 16

R = 128
SLAB = R * N
N_SLABS = N // R
ET = E // NS
ROWS128 = ET // 128
ZB = 4096
WB = SLAB // NS


def _coalesce_body(ei_hbm, ev_hbm, d_hbm,
                   row_b, col_b, g_b, v_b, idx_st, val_st, zero_b, dummy_b,
                   slab0, slab1, wb_sem0, wb_sem1, sc_sem, clr_sem):
    cid = lax.axis_index("c")
    sid = lax.axis_index("s")
    slabs = (slab0, slab1)
    wb_sems = (wb_sem0, wb_sem1)

    def _zero(i, _):
        zero_b[pl.ds(i * L, L)] = jnp.zeros((L,), jnp.float32)
        return 0
    lax.fori_loop(0, ZB // L, _zero, 0)

    def _wb_wait(half):
        pltpu.make_async_copy(slabs[half].at[pl.ds(sid * WB, WB)],
                              d_hbm.at[pl.ds(sid * WB, WB)],
                              wb_sems[half]).wait()

    def _fire_clears(half):
        for k in range(WB // ZB):
            pltpu.async_copy(zero_b, slabs[half].at[pl.ds(sid * WB + k * ZB,
                                                          ZB)], clr_sem)

    def _wait_clears(half):
        for k in range(WB // ZB):
            pltpu.make_async_copy(
                zero_b, slabs[half].at[pl.ds(sid * WB + k * ZB, ZB)],
                clr_sem).wait()

    _fire_clears(0)

    for ch_local in range(C_IN // NC):
        ch = cid * (C_IN // NC) + ch_local
        ebase = ch * 2 * E + sid * ET
        pltpu.sync_copy(ei_hbm.at[pl.ds(ebase, ET)], row_b)
        pltpu.sync_copy(ei_hbm.at[pl.ds(ebase + E, ET)], col_b)
        pltpu.sync_copy(ev_hbm.at[pl.ds(ch * E + sid * ET, ET)], v_b)

        def _flat(i, _):
            r = jnp.minimum(row_b[pl.ds(i * L, L)], N - 1)
            c = jnp.minimum(col_b[pl.ds(i * L, L)], N - 1)
            g_b[pl.ds(i * L, L)] = r * N + c
            return 0
        lax.fori_loop(0, ET // L, _flat, 0)

        def _slab_pair(s2, _):
            for half in range(2):
                s = s2 * 2 + half
                slab = slabs[half]

                _wait_clears(half)
                plsc.subcore_barrier()

                lo = s * SLAB

                def _stage(r128, _):
                    for k in range(128 // L):
                        i = r128 * (128 // L) + k
                        g = g_b[pl.ds(i * L, L)]
                        local = g - lo
                        inb = (local >= 0) & (local < SLAB)
                        idx_st[r128, pl.ds(k * L, L)] = lax.bitwise_and(
                            local, SLAB - 1)
                        v = v_b[pl.ds(i * L, L)]
                        val_st[r128, pl.ds(k * L, L)] = jnp.where(inb, v, 0.0)
                    pltpu.async_copy(val_st.at[r128],
                                     slab.at[idx_st.at[r128]], sc_sem,
                                     add=True)
                    return 0
                lax.fori_loop(0, ROWS128, _stage, 0)
                pltpu.make_async_copy(ei_hbm.at[pl.ds(0, ET)], dummy_b,
                                      sc_sem).wait()
                plsc.subcore_barrier()

                dst = ch * N * N + s * SLAB + sid * WB
                pltpu.async_copy(slab.at[pl.ds(sid * WB, WB)],
                                 d_hbm.at[pl.ds(dst, WB)], wb_sems[half])

                other = 1 - half
                if ch_local == 0 and half == 0:
                    @pl.when(s2 > 0)
                    def _():
                        _wb_wait(other)
                else:
                    _wb_wait(other)
                _fire_clears(other)
            return 0
        lax.fori_loop(0, N_SLABS // 2, _slab_pair, 0)
    _wait_clears(0)
    _wb_wait(1)


def _coalesce(edge_index, edge_value):
    mesh = plsc.VectorSubcoreMesh(core_axis_name="c", subcore_axis_name="s")
    kern = pl.kernel(
        _coalesce_body,
        out_type=jax.ShapeDtypeStruct((C_IN * N * N,), jnp.float32),
        mesh=mesh,
        scratch_types=[
            pltpu.VMEM((ET,), jnp.int32),
            pltpu.VMEM((ET,), jnp.int32),
            pltpu.VMEM((ET,), jnp.int32),
            pltpu.VMEM((ET,), jnp.float32),
            pltpu.VMEM((ROWS128, 128), jnp.int32),
            pltpu.VMEM((ROWS128, 128), jnp.float32),
            pltpu.VMEM((ZB,), jnp.float32),
            pltpu.VMEM((ET,), jnp.int32),
            pltpu.VMEM_SHARED((SLAB,), jnp.float32),
            pltpu.VMEM_SHARED((SLAB,), jnp.float32),
            pltpu.SemaphoreType.DMA,
            pltpu.SemaphoreType.DMA,
            pltpu.SemaphoreType.DMA,
            pltpu.SemaphoreType.DMA,
        ],
    )
    return kern(edge_index.reshape(-1), edge_value.reshape(-1))


MIX_BM = 256


def _mix_body(f1_ref, f2_ref, d_ref, a_ref, b_ref):
    d = d_ref[...]
    for c in range(C_OUT):
        acc_a = f1_ref[c, 0] * d[0]
        acc_b = f2_ref[c, 0] * d[0]
        for j in range(1, C_IN):
            acc_a += f1_ref[c, j] * d[j]
            acc_b += f2_ref[c, j] * d[j]
        a_ref[c] = acc_a.astype(jnp.bfloat16)
        b_ref[c] = acc_b.astype(jnp.bfloat16)


def _mix(filt1, filt2, d):
    grid = (N // MIX_BM,)
    return pl.pallas_call(
        _mix_body,
        grid=grid,
        in_specs=[
            pl.BlockSpec(memory_space=pltpu.SMEM),
            pl.BlockSpec(memory_space=pltpu.SMEM),
            pl.BlockSpec((C_IN, MIX_BM, N), lambda i: (0, i, 0)),
        ],
        out_specs=[
            pl.BlockSpec((C_OUT, MIX_BM, N), lambda i: (0, i, 0)),
            pl.BlockSpec((C_OUT, MIX_BM, N), lambda i: (0, i, 0)),
        ],
        out_shape=[
            jax.ShapeDtypeStruct((C_OUT, N, N), jnp.bfloat16),
            jax.ShapeDtypeStruct((C_OUT, N, N), jnp.bfloat16),
        ],
        compiler_params=pltpu.CompilerParams(
            dimension_semantics=("parallel",),
        ),
    )(filt1, filt2, d)


BM = 2048
BN = 2048
BK = 1024


def _mm_body(a_ref, b_ref, h_ref):
    k = pl.program_id(3)

    @pl.when(k == 0)
    def _():
        h_ref[...] = jnp.zeros_like(h_ref)

    h_ref[0] += jnp.dot(a_ref[0], b_ref[0], preferred_element_type=jnp.float32)


def _matmul(a, b):
    grid = (C_OUT, N // BM, N // BN, N // BK)
    return pl.pallas_call(
        _mm_body,
        grid=grid,
        in_specs=[
            pl.BlockSpec((1, BM, BK), lambda c, i, j, k: (c, i, k)),
            pl.BlockSpec((1, BK, BN), lambda c, i, j, k: (c, k, j)),
        ],
        out_specs=pl.BlockSpec((1, BM, BN), lambda c, i, j, k: (c, i, j)),
        out_shape=jax.ShapeDtypeStruct((C_OUT, N, N), jnp.float32),
        compiler_params=pltpu.CompilerParams(
            dimension_semantics=("parallel", "parallel", "parallel", "arbitrary"),
        ),
    )(a, b)


def kernel(edge_index, edge_value, num_nodes, W1, W2):
    filt1 = jax.nn.softmax(W1, axis=1)
    filt2 = jax.nn.softmax(W2, axis=1)
    d = _coalesce(edge_index, edge_value).reshape(C_IN, N, N)
    a, b = _mix(filt1, filt2, d)
    h = _matmul(a, b)
    return (h, filt1, filt2)
